# Initial kernel scaffold; baseline (speedup 1.0000x reference)
import jax
import jax.numpy as jnp
from jax.experimental import pallas as pl


def kernel(input_ids, token_type_ids, word_emb, pos_emb, type_emb, W, b, gamma, beta):
    raise NotImplementedError("write your pallas kernel here")


# speedup vs baseline: 6.5433x; 6.5433x over previous
import functools

import jax
import jax.numpy as jnp
from jax import lax
from jax.experimental import pallas as pl
from jax.experimental.pallas import tpu as pltpu
from jax.experimental.pallas import tpu_sc as plsc

VOCAB = 30522
EMB = 128
HID = 512
B = 128
S = 512
EPS = 1e-12

NC = 2
NS = 16
NW = NC * NS

N_ROWS = B * S
ROWS_PER_W = N_ROWS // NW
CHUNK = 512
N_CHUNKS = ROWS_PER_W // CHUNK


def _sc_gather(table_hbm, idx_hbm, out_hbm, idx_v, rows_v, sem):
    wid = lax.axis_index("s") * NC + lax.axis_index("c")
    base = wid * ROWS_PER_W
    pltpu.sync_copy(idx_hbm.at[pl.ds(base, ROWS_PER_W)], idx_v)
    for j in range(N_CHUNKS):
        pltpu.async_copy(
            table_hbm.at[idx_v.at[pl.ds(j * CHUNK, CHUNK)]], rows_v, sem
        ).wait()
        pltpu.sync_copy(rows_v, out_hbm.at[pl.ds(base + j * CHUNK, CHUNK)])


def _gather_rows(table, ids):
    gather = functools.partial(
        pl.kernel,
        out_type=jax.ShapeDtypeStruct((N_ROWS, EMB), jnp.float32),
        mesh=plsc.VectorSubcoreMesh(
            core_axis_name="c", subcore_axis_name="s", num_cores=NC
        ),
        scratch_types=[
            pltpu.VMEM((ROWS_PER_W,), jnp.int32),
            pltpu.VMEM((CHUNK, EMB), jnp.float32),
            pltpu.SemaphoreType.DMA,
        ],
    )(_sc_gather)
    return gather(table, ids)


BG = 8


def _tc_dense(e_ref, tt_ref, posb_ref, te_ref, gam_ref, bet_ref, w_ref, out_ref):
    e = e_ref[...]
    z = jnp.zeros((BG, 1, EMB), jnp.float32)
    left = jnp.concatenate([e[:, 1:, :], z], axis=1)
    right = jnp.concatenate([z, e[:, :-1, :]], axis=1)
    tri = jnp.concatenate([left, e, right], axis=2).reshape(BG * S, 3 * EMB)
    x = jnp.dot(tri, w_ref[...], preferred_element_type=jnp.float32)
    x = x.reshape(BG, S, HID)
    te = te_ref[...]
    tt = tt_ref[...]
    typ = te[0][None, None, :] + tt[:, :, None] * (te[1] - te[0])[None, None, :]
    emb = x + posb_ref[...][None, :, :] + typ
    mean = jnp.mean(emb, axis=-1, keepdims=True)
    cen = emb - mean
    var = jnp.mean(cen * cen, axis=-1, keepdims=True)
    norm = cen * lax.rsqrt(var + EPS)
    out_ref[...] = norm * gam_ref[...][0][None, None, :] + bet_ref[...][0][None, None, :]


def kernel(input_ids, token_type_ids, word_emb, pos_emb, type_emb, W, b, gamma, beta):
    ids = input_ids.reshape(-1).astype(jnp.int32)
    e = _gather_rows(word_emb, ids).reshape(B, S, EMB)

    tt_f = token_type_ids.astype(jnp.float32)
    posb = pos_emb + b[None, :]
    gam = gamma.reshape(1, HID)
    bet = beta.reshape(1, HID)

    grid = (B // BG,)
    out = pl.pallas_call(
        _tc_dense,
        grid=grid,
        in_specs=[
            pl.BlockSpec((BG, S, EMB), lambda i: (i, 0, 0)),
            pl.BlockSpec((BG, S), lambda i: (i, 0)),
            pl.BlockSpec((S, HID), lambda i: (0, 0)),
            pl.BlockSpec((2, HID), lambda i: (0, 0)),
            pl.BlockSpec((1, HID), lambda i: (0, 0)),
            pl.BlockSpec((1, HID), lambda i: (0, 0)),
            pl.BlockSpec((3 * EMB, HID), lambda i: (0, 0)),
        ],
        out_specs=pl.BlockSpec((BG, S, HID), lambda i: (i, 0, 0)),
        out_shape=jax.ShapeDtypeStruct((B, S, HID), jnp.float32),
    )(e, tt_f, posb, type_emb, gam, bet, W)
    return out


# speedup vs baseline: 6.5488x; 1.0009x over previous
import functools

import jax
import jax.numpy as jnp
from jax import lax
from jax.experimental import pallas as pl
from jax.experimental.pallas import tpu as pltpu
from jax.experimental.pallas import tpu_sc as plsc

VOCAB = 30522
EMB = 128
HID = 512
B = 128
S = 512
EPS = 1e-12

NC = 2
NS = 16
NW = NC * NS

N_ROWS = B * S
ROWS_PER_W = N_ROWS // NW
CHUNK = 256
N_CHUNKS = ROWS_PER_W // CHUNK


def _sc_gather(table_hbm, idx_hbm, out_hbm, idx_v, rows_v, gsem0, gsem1, ssem0, ssem1):
    wid = lax.axis_index("s") * NC + lax.axis_index("c")
    base = wid * ROWS_PER_W
    pltpu.sync_copy(idx_hbm.at[pl.ds(base, ROWS_PER_W)], idx_v)
    gsems = (gsem0, gsem1)
    ssems = (ssem0, ssem1)

    def gather_start(j, bb):
        return pltpu.async_copy(
            table_hbm.at[idx_v.at[pl.ds(j * CHUNK, CHUNK)]], rows_v.at[bb], gsems[bb]
        )

    g = [gather_start(0, 0), None]
    scat = [None, None]
    for j in range(N_CHUNKS):
        b = j & 1
        if j + 1 < N_CHUNKS:
            if scat[1 - b] is not None:
                scat[1 - b].wait()
            g[1 - b] = gather_start(j + 1, 1 - b)
        g[b].wait()
        scat[b] = pltpu.async_copy(
            rows_v.at[b], out_hbm.at[pl.ds(base + j * CHUNK, CHUNK)], ssems[b]
        )
    for b in (0, 1):
        if scat[b] is not None:
            scat[b].wait()


def _gather_rows(table, ids):
    gather = functools.partial(
        pl.kernel,
        out_type=jax.ShapeDtypeStruct((N_ROWS, EMB), jnp.float32),
        mesh=plsc.VectorSubcoreMesh(
            core_axis_name="c", subcore_axis_name="s", num_cores=NC
        ),
        scratch_types=[
            pltpu.VMEM((ROWS_PER_W,), jnp.int32),
            pltpu.VMEM((2, CHUNK, EMB), jnp.float32),
            pltpu.SemaphoreType.DMA,
            pltpu.SemaphoreType.DMA,
            pltpu.SemaphoreType.DMA,
            pltpu.SemaphoreType.DMA,
        ],
    )(_sc_gather)
    return gather(table, ids)


BG = 8


def _tc_dense(e_ref, tt_ref, posb_ref, te_ref, gam_ref, bet_ref, w_ref, out_ref):
    e = e_ref[...]
    z = jnp.zeros((BG, 1, EMB), jnp.float32)
    left = jnp.concatenate([e[:, 1:, :], z], axis=1)
    right = jnp.concatenate([z, e[:, :-1, :]], axis=1)
    tri = jnp.concatenate([left, e, right], axis=2).reshape(BG * S, 3 * EMB)
    x = jnp.dot(tri, w_ref[...], preferred_element_type=jnp.float32)
    x = x.reshape(BG, S, HID)
    te = te_ref[...]
    tt = tt_ref[...]
    typ = te[0][None, None, :] + tt[:, :, None] * (te[1] - te[0])[None, None, :]
    emb = x + posb_ref[...][None, :, :] + typ
    mean = jnp.mean(emb, axis=-1, keepdims=True)
    cen = emb - mean
    var = jnp.mean(cen * cen, axis=-1, keepdims=True)
    norm = cen * lax.rsqrt(var + EPS)
    out_ref[...] = norm * gam_ref[...][0][None, None, :] + bet_ref[...][0][None, None, :]


def kernel(input_ids, token_type_ids, word_emb, pos_emb, type_emb, W, b, gamma, beta):
    ids = input_ids.reshape(-1).astype(jnp.int32)
    e = _gather_rows(word_emb, ids).reshape(B, S, EMB)

    tt_f = token_type_ids.astype(jnp.float32)
    posb = pos_emb + b[None, :]
    gam = gamma.reshape(1, HID)
    bet = beta.reshape(1, HID)

    grid = (B // BG,)
    out = pl.pallas_call(
        _tc_dense,
        grid=grid,
        in_specs=[
            pl.BlockSpec((BG, S, EMB), lambda i: (i, 0, 0)),
            pl.BlockSpec((BG, S), lambda i: (i, 0)),
            pl.BlockSpec((S, HID), lambda i: (0, 0)),
            pl.BlockSpec((2, HID), lambda i: (0, 0)),
            pl.BlockSpec((1, HID), lambda i: (0, 0)),
            pl.BlockSpec((1, HID), lambda i: (0, 0)),
            pl.BlockSpec((3 * EMB, HID), lambda i: (0, 0)),
        ],
        out_specs=pl.BlockSpec((BG, S, HID), lambda i: (i, 0, 0)),
        out_shape=jax.ShapeDtypeStruct((B, S, HID), jnp.float32),
    )(e, tt_f, posb, type_emb, gam, bet, W)
    return out
